# Initial kernel scaffold; baseline (speedup 1.0000x reference)
import jax
import jax.numpy as jnp
from jax.experimental import pallas as pl


def kernel(faked_id, lookup_table):
    raise NotImplementedError("write your pallas kernel here")


# speedup vs baseline: 1.0438x; 1.0438x over previous
import functools

import jax
import jax.numpy as jnp
from jax import lax
from jax.experimental import pallas as pl
from jax.experimental.pallas import tpu as pltpu
from jax.experimental.pallas import tpu_sc as plsc

BATCH = 16384
SEQ_LEN = 7
TOTAL = BATCH * SEQ_LEN
NUM_WORKERS = 32
ROW = 128
NROWS = TOTAL // (NUM_WORKERS * ROW)


PER_W = NROWS * ROW


def _gather_kernel(idx_hbm, table_hbm, out_hbm, idx_v, rows_v, sem):
    wid = lax.axis_index("s") * 2 + lax.axis_index("c")
    base = wid * PER_W
    pltpu.sync_copy(idx_hbm.at[pl.ds(base, PER_W)], idx_v)
    copies = [
        pltpu.async_copy(
            table_hbm.at[idx_v.at[pl.ds(j * ROW, ROW)]],
            rows_v.at[pl.ds(j * ROW, ROW)],
            sem,
        )
        for j in range(NROWS)
    ]
    for c in copies:
        c.wait()
    pltpu.sync_copy(rows_v, out_hbm.at[pl.ds(base, PER_W)])


@jax.jit
def _run(idx_flat, table):
    mesh = plsc.VectorSubcoreMesh(core_axis_name="c", subcore_axis_name="s")
    fn = functools.partial(
        pl.kernel,
        out_type=jax.ShapeDtypeStruct((TOTAL,), jnp.int32),
        mesh=mesh,
        scratch_types=[
            pltpu.VMEM((PER_W,), jnp.int32),
            pltpu.VMEM((PER_W,), jnp.int32),
            pltpu.SemaphoreType.DMA,
        ],
    )(_gather_kernel)
    return fn(idx_flat, table)


def kernel(faked_id, lookup_table):
    idx_flat = faked_id.reshape(TOTAL)
    out = _run(idx_flat, lookup_table)
    return out.reshape(BATCH, SEQ_LEN)


# speedup vs baseline: 1.3537x; 1.2968x over previous
import functools

import jax
import jax.numpy as jnp
from jax import lax
from jax.experimental import pallas as pl
from jax.experimental.pallas import tpu as pltpu
from jax.experimental.pallas import tpu_sc as plsc

BATCH = 16384
SEQ_LEN = 7
NUM_WORKERS = 32
ROWS_W = BATCH // NUM_WORKERS
PER_W = ROWS_W * SEQ_LEN
ROW = 128
NCHUNK = PER_W // ROW
LANES = 16
NVEC = PER_W // LANES
RECIP7 = 9363


def _rc(t):
    k = t * LANES + lax.iota(jnp.int32, LANES)
    r = lax.shift_right_logical(k * RECIP7, 16)
    c = k - r * SEQ_LEN
    return r, c


def _gather_kernel(idx_hbm, table_hbm, out_hbm, pad_v, comp_v, res_v, sem):
    wid = lax.axis_index("s") * 2 + lax.axis_index("c")
    base = wid * ROWS_W
    pltpu.sync_copy(idx_hbm.at[pl.ds(base, ROWS_W), :], pad_v)

    def compact(t, carry):
        r, c = _rc(t)
        comp_v[pl.ds(t * LANES, LANES)] = plsc.load_gather(pad_v, [r, c])
        return carry

    lax.fori_loop(0, NVEC, compact, 0)

    copies = [
        pltpu.async_copy(
            table_hbm.at[comp_v.at[pl.ds(j * ROW, ROW)]],
            res_v.at[pl.ds(j * ROW, ROW)],
            sem,
        )
        for j in range(NCHUNK)
    ]
    for cp in copies:
        cp.wait()

    def expand(t, carry):
        r, c = _rc(t)
        plsc.store_scatter(pad_v, [r, c], res_v[pl.ds(t * LANES, LANES)])
        return carry

    lax.fori_loop(0, NVEC, expand, 0)
    pltpu.sync_copy(pad_v, out_hbm.at[pl.ds(base, ROWS_W), :])


@jax.jit
def _run(faked_id, table):
    mesh = plsc.VectorSubcoreMesh(core_axis_name="c", subcore_axis_name="s")
    fn = functools.partial(
        pl.kernel,
        out_type=jax.ShapeDtypeStruct((BATCH, SEQ_LEN), jnp.int32),
        mesh=mesh,
        compiler_params=pltpu.CompilerParams(needs_layout_passes=False),
        scratch_types=[
            pltpu.VMEM((ROWS_W, SEQ_LEN), jnp.int32),
            pltpu.VMEM((PER_W,), jnp.int32),
            pltpu.VMEM((PER_W,), jnp.int32),
            pltpu.SemaphoreType.DMA,
        ],
    )(_gather_kernel)
    return fn(faked_id, table)


def kernel(faked_id, lookup_table):
    return _run(faked_id, lookup_table)


# speedup vs baseline: 1.9877x; 1.4684x over previous
import functools

import jax
import jax.numpy as jnp
from jax import lax
from jax.experimental import pallas as pl
from jax.experimental.pallas import tpu as pltpu
from jax.experimental.pallas import tpu_sc as plsc

BATCH = 16384
SEQ_LEN = 7
NUM_WORKERS = 32
COLS_W = BATCH // NUM_WORKERS
ROW = 128
NCHUNK = COLS_W // ROW


PER_W = SEQ_LEN * COLS_W
CHUNKS = [(r, j) for r in range(SEQ_LEN) for j in range(NCHUNK)]


def _gather_kernel(idx_hbm, table_hbm, out_hbm, idx_v, res_v, sem, sem2):
    wid = lax.axis_index("s") * 2 + lax.axis_index("c")
    base = wid * COLS_W

    stages = [
        pltpu.async_copy(
            idx_hbm.at[pl.ds(r, 1), pl.ds(base + j * ROW, ROW)],
            idx_v.at[:, pl.ds((r * NCHUNK + j) * ROW, ROW)],
            sem2,
        )
        for r, j in CHUNKS
    ]
    for cp in stages:
        cp.wait()

    gathers = [
        pltpu.async_copy(
            table_hbm.at[idx_v.at[0, pl.ds(q * ROW, ROW)]],
            res_v.at[0, pl.ds(q * ROW, ROW)],
            sem,
        )
        for q in range(len(CHUNKS))
    ]
    for cp in gathers:
        cp.wait()

    writes = [
        pltpu.async_copy(
            res_v.at[:, pl.ds((r * NCHUNK + j) * ROW, ROW)],
            out_hbm.at[pl.ds(r, 1), pl.ds(base + j * ROW, ROW)],
            sem2,
        )
        for r, j in CHUNKS
    ]
    for cp in writes:
        cp.wait()


@jax.jit
def _run(idx_t, table):
    mesh = plsc.VectorSubcoreMesh(core_axis_name="c", subcore_axis_name="s")
    fn = functools.partial(
        pl.kernel,
        out_type=jax.ShapeDtypeStruct((SEQ_LEN, BATCH), jnp.int32),
        mesh=mesh,
        scratch_types=[
            pltpu.VMEM((1, PER_W), jnp.int32),
            pltpu.VMEM((1, PER_W), jnp.int32),
            pltpu.SemaphoreType.DMA,
            pltpu.SemaphoreType.DMA,
        ],
    )(_gather_kernel)
    return fn(idx_t, table)


def kernel(faked_id, lookup_table):
    out_t = _run(faked_id.T, lookup_table)
    return out_t.T
